# baseline (speedup 1.0000x reference)
import functools

import jax
import jax.numpy as jnp
from jax import lax
from jax.experimental import pallas as pl
from jax.experimental.pallas import tpu as pltpu
from jax.experimental.pallas import tpu_sc as plsc

NC = 2
NS = 16
NW = NC * NS
LANES = 16
NBUF = 3


def _mesh():
    return plsc.VectorSubcoreMesh(core_axis_name="c", subcore_axis_name="s")


def _make_deg_kernel(n_acc, ce, k_chunks):
    @functools.partial(
        pl.kernel,
        out_type=jax.ShapeDtypeStruct((NC, NS, n_acc), jnp.float32),
        mesh=_mesh(),
        scratch_types=[
            pltpu.VMEM((k_chunks, ce), jnp.int32),
            pltpu.VMEM((n_acc,), jnp.float32),
        ],
        compiler_params=pltpu.CompilerParams(needs_layout_passes=False),
    )
    def deg_kernel(dst_hbm, out_hbm, dst_v, hist_v):
        c = lax.axis_index("c")
        s = lax.axis_index("s")
        wid = s * NC + c

        def zero_body(i, _):
            hist_v[pl.ds(i * LANES, LANES)] = jnp.zeros((LANES,), jnp.float32)
            return 0

        lax.fori_loop(0, n_acc // LANES, zero_body, 0)

        pltpu.sync_copy(dst_hbm.at[wid], dst_v)
        ones = jnp.ones((LANES,), jnp.float32)

        def chunk_body(j, _):
            for k in range(ce // LANES):
                idx = dst_v[j, pl.ds(k * LANES, LANES)]
                plsc.addupdate_scatter(hist_v, [idx], ones)
            return 0

        lax.fori_loop(0, k_chunks, chunk_body, 0)
        pltpu.sync_copy(hist_v, out_hbm.at[c, s])

    return deg_kernel


def _zero_rows_buf(rows_v, ce, d):
    zz = jnp.zeros((LANES,), jnp.float32)

    def zrow(r, _):
        for k in range(d // LANES):
            rows_v[0, r, pl.ds(k * LANES, LANES)] = zz
        return 0

    lax.fori_loop(0, ce, zrow, 0)


def _zero_acc_slice(rows_v, acc_sh, r0, rows_per_tile, ce):
    nfull = rows_per_tile // ce
    tail = rows_per_tile % ce

    def zcp(b, _):
        pltpu.sync_copy(rows_v.at[0], acc_sh.at[pl.ds(r0 + b * ce, ce)])
        return 0

    lax.fori_loop(0, nfull, zcp, 0)
    if tail:
        pltpu.sync_copy(rows_v.at[0, pl.ds(0, tail)],
                        acc_sh.at[pl.ds(r0 + nfull * ce, tail)])


def _make_prop1_fused(n_acc, n_acc2, d, ce, ib, k_chunks):
    rows_per_tile = n_acc // NS
    blk = n_acc2 // NS
    hrows = n_acc2 // LANES
    groups = k_chunks // ib

    @functools.partial(
        pl.kernel,
        out_type=[
            jax.ShapeDtypeStruct((NC, n_acc, d), jnp.float32),
            jax.ShapeDtypeStruct((n_acc2,), jnp.float32),
            jax.ShapeDtypeStruct((NC, n_acc2, d), jnp.float32),
        ],
        mesh=_mesh(),
        scratch_types=[
            pltpu.VMEM((ib, ce), jnp.int32),
            pltpu.VMEM((ib, ce), jnp.int32),
            pltpu.VMEM((NBUF, ce, d), jnp.float32),
            pltpu.VMEM((hrows, LANES), jnp.float32),
            pltpu.VMEM((blk,), jnp.float32),
            pltpu.VMEM((5, 128), jnp.int32),
            pltpu.VMEM_SHARED((n_acc, d), jnp.float32),
            pltpu.VMEM_SHARED((hrows, LANES), jnp.float32),
        ] + [pltpu.SemaphoreType.DMA] * (2 * NBUF),
        compiler_params=pltpu.CompilerParams(use_tc_tiling_on_sc=False,
                                             needs_layout_passes=False),
    )
    def prop1_kernel(nodes_hbm, src_hbm, dst_hbm,
                     p_out, dinv_out, xs_hbm,
                     src_v, dst_v, rows_v, hist_v, dinv_v, idx5_v,
                     acc_sh, deg_sh, *sems):
        c = lax.axis_index("c")
        s = lax.axis_index("s")
        wid = s * NC + c
        r0 = s * rows_per_tile
        b0 = s * blk
        h0 = s * (hrows // NS)
        gsem = sems[:NBUF]
        ssem = sems[NBUF:]
        magic = jnp.full((LANES,), 0x5F3759DF, jnp.int32)
        c15 = jnp.full((LANES,), 1.5, jnp.float32)
        c05 = jnp.full((LANES,), 0.5, jnp.float32)
        zzf = jnp.zeros((LANES,), jnp.float32)
        ones = jnp.ones((LANES,), jnp.float32)
        fifteen = jnp.full((LANES,), 15, jnp.int32)

        _zero_rows_buf(rows_v, ce, d)
        _zero_acc_slice(rows_v, acc_sh, r0, rows_per_tile, ce)
        pltpu.sync_copy(rows_v.at[0, pl.ds(0, hrows // NS), pl.ds(0, LANES)],
                        deg_sh.at[pl.ds(h0, hrows // NS)])

        def zh(r, _):
            hist_v[r, :] = zzf
            return 0

        lax.fori_loop(0, hrows, zh, 0)
        for w01 in range(2):
            def hg(g, _):
                pltpu.sync_copy(
                    dst_hbm.at[2 * s + w01, pl.ds(g * ib, ib)], dst_v)
                for j in range(ib):
                    for k in range(ce // LANES):
                        idx = dst_v[j, pl.ds(k * LANES, LANES)]
                        ri = lax.shift_right_logical(idx, 4)
                        ci = lax.bitwise_and(idx, fifteen)
                        plsc.addupdate_scatter(hist_v, [ri, ci], ones)
                return 0

            lax.fori_loop(0, groups, hg, 0)

        for m in range(40):
            idx5_v[m // 8, pl.ds((m % 8) * LANES, LANES)] = (
                lax.iota(jnp.int32, LANES) + m * LANES)
        plsc.subcore_barrier()
        for r in range(5):
            pltpu.sync_copy(hist_v.at[pl.ds(r * 128, 128)],
                            deg_sh.at[idx5_v.at[r]], add=True)
        plsc.subcore_barrier()

        pltpu.sync_copy(deg_sh.at[pl.ds(h0, hrows // NS)],
                        hist_v.at[pl.ds(0, hrows // NS)])
        for m in range(hrows // NS):
            x = hist_v[m, :]
            i = plsc.bitcast(x, jnp.int32)
            y = plsc.bitcast(magic - lax.shift_right_logical(i, 1),
                             jnp.float32)
            hx = c05 * x
            for _ in range(3):
                y = y * (c15 - hx * y * y)
            dinv_v[pl.ds(m * LANES, LANES)] = jnp.where(x > 0.0, y, zzf)

        @pl.when(c == 0)
        def _():
            pltpu.sync_copy(dinv_v, dinv_out.at[pl.ds(b0, blk)])

        def presc(u, _):
            st = rows_v.at[0, pl.ds(0, 64)]
            pltpu.sync_copy(nodes_hbm.at[pl.ds(b0 + u * 64, 64)], st)
            for r in range(64):
                v = u * 64 + r
                dv = plsc.load_gather(
                    dinv_v, [jnp.full((LANES,), v, jnp.int32)])
                for k in range(d // LANES):
                    sl = pl.ds(k * LANES, LANES)
                    rows_v[0, r, sl] = rows_v[0, r, sl] * dv
            pltpu.sync_copy(st, xs_hbm.at[c, pl.ds(b0 + u * 64, 64)])
            return 0

        lax.fori_loop(0, blk // 64, presc, 0)
        plsc.subcore_barrier()

        xs_c = xs_hbm.at[c]

        def wait_gather(b):
            pltpu.make_async_copy(xs_c.at[src_v.at[0]], rows_v.at[b],
                                  gsem[b]).wait()

        def wait_scatter(b):
            pltpu.make_async_copy(rows_v.at[b], acc_sh.at[dst_v.at[0]],
                                  ssem[b]).wait()

        def group_body(g, _):
            pltpu.sync_copy(src_hbm.at[wid, pl.ds(g * ib, ib)], src_v)
            pltpu.sync_copy(dst_hbm.at[wid, pl.ds(g * ib, ib)], dst_v)
            for j in range(ib):
                b = j % NBUF
                if j >= NBUF:
                    wait_scatter(b)
                pltpu.async_copy(xs_c.at[src_v.at[j]], rows_v.at[b],
                                 gsem[b])
                if j >= 2:
                    bp = (j - 2) % NBUF
                    wait_gather(bp)
                    pltpu.async_copy(rows_v.at[bp],
                                     acc_sh.at[dst_v.at[j - 2]],
                                     ssem[bp], add=True)
            for t in (ib - 2, ib - 1):
                bp = t % NBUF
                wait_gather(bp)
                pltpu.async_copy(rows_v.at[bp], acc_sh.at[dst_v.at[t]],
                                 ssem[bp], add=True)
            for t in (ib - 3, ib - 2, ib - 1):
                wait_scatter(t % NBUF)
            return 0

        lax.fori_loop(0, groups, group_body, 0)
        plsc.subcore_barrier()
        pltpu.sync_copy(acc_sh.at[pl.ds(r0, rows_per_tile)],
                        p_out.at[c, pl.ds(r0, rows_per_tile)])

    return prop1_kernel


def _make_prop_kernel(n_acc, d, ce, ib, k_chunks):
    rows_per_tile = n_acc // NS
    groups = k_chunks // ib

    @functools.partial(
        pl.kernel,
        out_type=jax.ShapeDtypeStruct((NC, n_acc, d), jnp.float32),
        mesh=_mesh(),
        scratch_types=[
            pltpu.VMEM((ib, ce), jnp.int32),
            pltpu.VMEM((ib, ce), jnp.int32),
            pltpu.VMEM((NBUF, ce, d), jnp.float32),
            pltpu.VMEM_SHARED((n_acc, d), jnp.float32),
        ] + [pltpu.SemaphoreType.DMA] * (2 * NBUF),
        compiler_params=pltpu.CompilerParams(use_tc_tiling_on_sc=False),
    )
    def prop_kernel(xs_hbm, src_hbm, dst_hbm, out_hbm,
                    src_v, dst_v, rows_v, acc_sh, *sems):
        c = lax.axis_index("c")
        s = lax.axis_index("s")
        wid = s * NC + c
        r0 = s * rows_per_tile
        gsem = sems[:NBUF]
        ssem = sems[NBUF:]

        _zero_rows_buf(rows_v, ce, d)
        _zero_acc_slice(rows_v, acc_sh, r0, rows_per_tile, ce)
        plsc.subcore_barrier()

        def wait_gather(b):
            pltpu.make_async_copy(xs_hbm.at[src_v.at[0]], rows_v.at[b],
                                  gsem[b]).wait()

        def wait_scatter(b):
            pltpu.make_async_copy(rows_v.at[b], acc_sh.at[dst_v.at[0]],
                                  ssem[b]).wait()

        def group_body(g, _):
            pltpu.sync_copy(src_hbm.at[wid, pl.ds(g * ib, ib)], src_v)
            pltpu.sync_copy(dst_hbm.at[wid, pl.ds(g * ib, ib)], dst_v)
            for j in range(ib):
                b = j % NBUF
                if j >= NBUF:
                    wait_scatter(b)
                pltpu.async_copy(xs_hbm.at[src_v.at[j]], rows_v.at[b],
                                 gsem[b])
                if j >= 2:
                    bp = (j - 2) % NBUF
                    wait_gather(bp)
                    pltpu.async_copy(rows_v.at[bp],
                                     acc_sh.at[dst_v.at[j - 2]],
                                     ssem[bp], add=True)
            for t in (ib - 2, ib - 1):
                bp = t % NBUF
                wait_gather(bp)
                pltpu.async_copy(rows_v.at[bp], acc_sh.at[dst_v.at[t]],
                                 ssem[bp], add=True)
            for t in (ib - 3, ib - 2, ib - 1):
                wait_scatter(t % NBUF)
            return 0

        lax.fori_loop(0, groups, group_body, 0)
        plsc.subcore_barrier()
        pltpu.sync_copy(acc_sh.at[pl.ds(r0, rows_per_tile)],
                        out_hbm.at[c, pl.ds(r0, rows_per_tile)])

    return prop_kernel


def _tc_a_body(deg_ref, nodes_ref, xs_ref, dinv_ref):
    deg = jnp.sum(deg_ref[...], axis=(0, 1))[:, None]
    dinv = jnp.where(deg > 0.0, lax.rsqrt(jnp.maximum(deg, 1e-12)), 0.0)
    dinv_ref[...] = dinv
    xs_ref[...] = nodes_ref[...] * dinv


def _tc_b_body(p_ref, dinv_ref, w1_ref, b1_ref, w2_ref, gs_ref):
    dinv = dinv_ref[...]
    x = (p_ref[0] + p_ref[1]) * dinv
    h = jnp.dot(x, w1_ref[...], preferred_element_type=jnp.float32,
                precision=lax.Precision.HIGHEST) + b1_ref[...]
    h = jnp.maximum(h, 0.0)
    g = jnp.dot(h, w2_ref[...], preferred_element_type=jnp.float32,
                precision=lax.Precision.HIGHEST)
    gs_ref[...] = g * dinv


def _tc_c_body(q_ref, dinv_ref, b2_ref, w3_ref, b3_ref, w4_ref, b4_ref,
               out_ref):
    x2 = jnp.maximum((q_ref[0] + q_ref[1]) * dinv_ref[...] + b2_ref[...], 0.0)
    x3 = jnp.dot(x2, w3_ref[...], preferred_element_type=jnp.float32,
                 precision=lax.Precision.HIGHEST) + b3_ref[...]
    x3 = jnp.maximum(x3, 0.0)
    x4 = jnp.dot(x3, w4_ref[...], preferred_element_type=jnp.float32,
                 precision=lax.Precision.HIGHEST) + b4_ref[...]
    out_ref[...] = jax.nn.sigmoid(x4)


def kernel(nodes, edges, W1, b1, W2, b2, W3, b3, W4, b4):
    n = nodes.shape[0]
    d_in = nodes.shape[1]
    e = edges.shape[1]
    h1 = W1.shape[1]
    d2 = 160
    n_acc = ((n + 8 + NS - 1) // NS) * NS

    n_acc2 = ((n_acc + NS * 64 - 1) // (NS * 64)) * (NS * 64)

    ce1, ib1 = 64, 54
    ce2, ib2 = 48, 72
    e2 = e + n
    per_w = (e2 + NW * ce1 * ib1 - 1) // (NW * ce1 * ib1) * (ce1 * ib1)
    k1 = per_w // ce1
    k2 = per_w // ce2
    e_pad = NW * per_w
    npad = e_pad - e2
    loop_idx = jnp.arange(n, dtype=jnp.int32)
    pad_src = jnp.arange(npad, dtype=jnp.int32) % n
    pad_dst = n + (jnp.arange(npad, dtype=jnp.int32) % 8)
    src_all = jnp.concatenate([edges[0], loop_idx, pad_src])
    dst_all = jnp.concatenate([edges[1], loop_idx, pad_dst])
    src_r1 = src_all.reshape(NW, k1, ce1)
    dst_r1 = dst_all.reshape(NW, k1, ce1)
    src_r2 = src_all.reshape(NW, k2, ce2)
    dst_r2 = dst_all.reshape(NW, k2, ce2)

    nodes_p = jnp.pad(nodes, ((0, n_acc2 - n), (0, 0)))
    W2p = jnp.pad(W2, ((0, 0), (0, d2 - W2.shape[1])))
    b2p = jnp.pad(b2, (0, d2 - b2.shape[0])).reshape(1, d2)
    W3p = jnp.pad(W3, ((0, d2 - W3.shape[0]), (0, 0)))
    b1r = b1.reshape(1, h1)
    b3r = b3.reshape(1, W3.shape[1])
    b4r = b4.reshape(1, 1)

    p, dinv_flat, _ = _make_prop1_fused(n_acc, n_acc2, d_in, ce1, ib1, k1)(
        nodes_p, src_r1, dst_r1)
    dinv = dinv_flat[:n_acc].reshape(n_acc, 1)

    nblk = 4
    r = n_acc // nblk
    full = lambda shape: pl.BlockSpec(shape, lambda i: (0,) * len(shape))
    gs = pl.pallas_call(
        _tc_b_body,
        grid=(nblk,),
        in_specs=[
            pl.BlockSpec((NC, r, d_in), lambda i: (0, i, 0)),
            pl.BlockSpec((r, 1), lambda i: (i, 0)),
            full((d_in, h1)),
            full((1, h1)),
            full((h1, d2)),
        ],
        out_specs=pl.BlockSpec((r, d2), lambda i: (i, 0)),
        out_shape=jax.ShapeDtypeStruct((n_acc, d2), jnp.float32),
    )(p, dinv, W1, b1r, W2p)

    q = _make_prop_kernel(n_acc, d2, ce2, ib2, k2)(gs, src_r2, dst_r2)

    h3 = W3.shape[1]
    out = pl.pallas_call(
        _tc_c_body,
        grid=(nblk,),
        in_specs=[
            pl.BlockSpec((NC, r, d2), lambda i: (0, i, 0)),
            pl.BlockSpec((r, 1), lambda i: (i, 0)),
            full((1, d2)),
            full((d2, h3)),
            full((1, h3)),
            full((h3, 1)),
            full((1, 1)),
        ],
        out_specs=pl.BlockSpec((r, 1), lambda i: (i, 0)),
        out_shape=jax.ShapeDtypeStruct((n_acc, 1), jnp.float32),
    )(q, dinv, b2p, W3p, b3r, W4, b4r)

    return out[:n]


# speedup vs baseline: 1.0708x; 1.0708x over previous
import functools

import jax
import jax.numpy as jnp
from jax import lax
from jax.experimental import pallas as pl
from jax.experimental.pallas import tpu as pltpu
from jax.experimental.pallas import tpu_sc as plsc

NC = 2
NS = 16
NW = NC * NS
LANES = 16
NBUF = 3


def _mesh():
    return plsc.VectorSubcoreMesh(core_axis_name="c", subcore_axis_name="s")


def _make_deg_kernel(n_acc, ce, k_chunks):
    @functools.partial(
        pl.kernel,
        out_type=jax.ShapeDtypeStruct((NC, NS, n_acc), jnp.float32),
        mesh=_mesh(),
        scratch_types=[
            pltpu.VMEM((k_chunks, ce), jnp.int32),
            pltpu.VMEM((n_acc,), jnp.float32),
        ],
        compiler_params=pltpu.CompilerParams(needs_layout_passes=False),
    )
    def deg_kernel(dst_hbm, out_hbm, dst_v, hist_v):
        c = lax.axis_index("c")
        s = lax.axis_index("s")
        wid = s * NC + c

        def zero_body(i, _):
            hist_v[pl.ds(i * LANES, LANES)] = jnp.zeros((LANES,), jnp.float32)
            return 0

        lax.fori_loop(0, n_acc // LANES, zero_body, 0)

        pltpu.sync_copy(dst_hbm.at[wid], dst_v)
        ones = jnp.ones((LANES,), jnp.float32)

        def chunk_body(j, _):
            for k in range(ce // LANES):
                idx = dst_v[j, pl.ds(k * LANES, LANES)]
                plsc.addupdate_scatter(hist_v, [idx], ones)
            return 0

        lax.fori_loop(0, k_chunks, chunk_body, 0)
        pltpu.sync_copy(hist_v, out_hbm.at[c, s])

    return deg_kernel


def _make_prop_kernel(n_acc, d, ce, ib, k_chunks):
    rows_per_tile = n_acc // NS
    groups = k_chunks // ib

    @functools.partial(
        pl.kernel,
        out_type=jax.ShapeDtypeStruct((NC, n_acc, d), jnp.float32),
        mesh=_mesh(),
        scratch_types=[
            pltpu.VMEM((ib, ce), jnp.int32),
            pltpu.VMEM((ib, ce), jnp.int32),
            pltpu.VMEM((NBUF, ce, d), jnp.float32),
            pltpu.VMEM_SHARED((n_acc, d), jnp.float32),
        ] + [pltpu.SemaphoreType.DMA] * (2 * NBUF),
        compiler_params=pltpu.CompilerParams(use_tc_tiling_on_sc=False),
    )
    def prop_kernel(xs_hbm, src_hbm, dst_hbm, zeros_hbm, out_hbm,
                    src_v, dst_v, rows_v, acc_sh, *sems):
        c = lax.axis_index("c")
        s = lax.axis_index("s")
        wid = s * NC + c
        r0 = s * rows_per_tile
        gsem = sems[:NBUF]
        ssem = sems[NBUF:]

        pltpu.sync_copy(zeros_hbm.at[pl.ds(r0, rows_per_tile)],
                        acc_sh.at[pl.ds(r0, rows_per_tile)])
        plsc.subcore_barrier()

        def wait_gather(b):
            pltpu.make_async_copy(xs_hbm.at[src_v.at[0]], rows_v.at[b],
                                  gsem[b]).wait()

        def wait_scatter(b):
            pltpu.make_async_copy(rows_v.at[b], acc_sh.at[dst_v.at[0]],
                                  ssem[b]).wait()

        def group_body(g, _):
            pltpu.sync_copy(src_hbm.at[wid, pl.ds(g * ib, ib)], src_v)
            pltpu.sync_copy(dst_hbm.at[wid, pl.ds(g * ib, ib)], dst_v)
            for j in range(ib):
                b = j % NBUF
                if j >= NBUF:
                    wait_scatter(b)
                pltpu.async_copy(xs_hbm.at[src_v.at[j]], rows_v.at[b],
                                 gsem[b])
                if j >= 2:
                    bp = (j - 2) % NBUF
                    wait_gather(bp)
                    pltpu.async_copy(rows_v.at[bp],
                                     acc_sh.at[dst_v.at[j - 2]],
                                     ssem[bp], add=True)
            for t in (ib - 2, ib - 1):
                bp = t % NBUF
                wait_gather(bp)
                pltpu.async_copy(rows_v.at[bp], acc_sh.at[dst_v.at[t]],
                                 ssem[bp], add=True)
            for t in (ib - 3, ib - 2, ib - 1):
                wait_scatter(t % NBUF)
            return 0

        lax.fori_loop(0, groups, group_body, 0)
        plsc.subcore_barrier()
        pltpu.sync_copy(acc_sh.at[pl.ds(r0, rows_per_tile)],
                        out_hbm.at[c, pl.ds(r0, rows_per_tile)])

    return prop_kernel


def _tc_a_body(deg_ref, nodes_ref, xs_ref, dinv_ref):
    deg = jnp.sum(deg_ref[...], axis=(0, 1))[:, None]
    dinv = jnp.where(deg > 0.0, lax.rsqrt(jnp.maximum(deg, 1e-12)), 0.0)
    dinv_ref[...] = dinv
    xs_ref[...] = nodes_ref[...] * dinv


def _tc_b_body(p_ref, dinv_ref, w1_ref, b1_ref, w2_ref, gs_ref):
    dinv = dinv_ref[...]
    x = (p_ref[0] + p_ref[1]) * dinv
    h = jnp.dot(x, w1_ref[...], preferred_element_type=jnp.float32,
                precision=lax.Precision.HIGHEST) + b1_ref[...]
    h = jnp.maximum(h, 0.0)
    g = jnp.dot(h, w2_ref[...], preferred_element_type=jnp.float32,
                precision=lax.Precision.HIGHEST)
    gs_ref[...] = g * dinv


def _tc_c_body(q_ref, dinv_ref, b2_ref, w3_ref, b3_ref, w4_ref, b4_ref,
               out_ref):
    x2 = jnp.maximum((q_ref[0] + q_ref[1]) * dinv_ref[...] + b2_ref[...], 0.0)
    x3 = jnp.dot(x2, w3_ref[...], preferred_element_type=jnp.float32,
                 precision=lax.Precision.HIGHEST) + b3_ref[...]
    x3 = jnp.maximum(x3, 0.0)
    x4 = jnp.dot(x3, w4_ref[...], preferred_element_type=jnp.float32,
                 precision=lax.Precision.HIGHEST) + b4_ref[...]
    out_ref[...] = jax.nn.sigmoid(x4)


def kernel(nodes, edges, W1, b1, W2, b2, W3, b3, W4, b4):
    n = nodes.shape[0]
    d_in = nodes.shape[1]
    e = edges.shape[1]
    h1 = W1.shape[1]
    d2 = 160
    n_acc = ((n + 8 + NS - 1) // NS) * NS

    ce1, ib1 = 96, 36
    ce2, ib2 = 48, 72
    e2 = e + n
    per_w = (e2 + NW * ce1 * ib1 - 1) // (NW * ce1 * ib1) * (ce1 * ib1)
    k1 = per_w // ce1
    k2 = per_w // ce2
    e_pad = NW * per_w
    npad = e_pad - e2
    loop_idx = jnp.arange(n, dtype=jnp.int32)
    pad_src = jnp.arange(npad, dtype=jnp.int32) % n
    pad_dst = n + (jnp.arange(npad, dtype=jnp.int32) % 8)
    src_all = jnp.concatenate([edges[0], loop_idx, pad_src])
    dst_all = jnp.concatenate([edges[1], loop_idx, pad_dst])
    src_r1 = src_all.reshape(NW, k1, ce1)
    dst_r1 = dst_all.reshape(NW, k1, ce1)
    src_r2 = src_all.reshape(NW, k2, ce2)
    dst_r2 = dst_all.reshape(NW, k2, ce2)

    nodes_p = jnp.pad(nodes, ((0, n_acc - n), (0, 0)))
    zeros1 = jnp.zeros((n_acc, d_in), jnp.float32)
    zeros2 = jnp.zeros((n_acc, d2), jnp.float32)
    W2p = jnp.pad(W2, ((0, 0), (0, d2 - W2.shape[1])))
    b2p = jnp.pad(b2, (0, d2 - b2.shape[0])).reshape(1, d2)
    W3p = jnp.pad(W3, ((0, d2 - W3.shape[0]), (0, 0)))
    b1r = b1.reshape(1, h1)
    b3r = b3.reshape(1, W3.shape[1])
    b4r = b4.reshape(1, 1)

    deg_parts = _make_deg_kernel(n_acc, ce2, k2)(dst_r2)

    xs1, dinv = pl.pallas_call(
        _tc_a_body,
        out_shape=[
            jax.ShapeDtypeStruct((n_acc, d_in), jnp.float32),
            jax.ShapeDtypeStruct((n_acc, 1), jnp.float32),
        ],
    )(deg_parts, nodes_p)

    p = _make_prop_kernel(n_acc, d_in, ce1, ib1, k1)(
        xs1, src_r1, dst_r1, zeros1)

    nblk = 4
    r = n_acc // nblk
    full = lambda shape: pl.BlockSpec(shape, lambda i: (0,) * len(shape))
    gs = pl.pallas_call(
        _tc_b_body,
        grid=(nblk,),
        in_specs=[
            pl.BlockSpec((NC, r, d_in), lambda i: (0, i, 0)),
            pl.BlockSpec((r, 1), lambda i: (i, 0)),
            full((d_in, h1)),
            full((1, h1)),
            full((h1, d2)),
        ],
        out_specs=pl.BlockSpec((r, d2), lambda i: (i, 0)),
        out_shape=jax.ShapeDtypeStruct((n_acc, d2), jnp.float32),
    )(p, dinv, W1, b1r, W2p)

    q = _make_prop_kernel(n_acc, d2, ce2, ib2, k2)(gs, src_r2, dst_r2, zeros2)

    h3 = W3.shape[1]
    out = pl.pallas_call(
        _tc_c_body,
        grid=(nblk,),
        in_specs=[
            pl.BlockSpec((NC, r, d2), lambda i: (0, i, 0)),
            pl.BlockSpec((r, 1), lambda i: (i, 0)),
            full((1, d2)),
            full((d2, h3)),
            full((1, h3)),
            full((h3, 1)),
            full((1, 1)),
        ],
        out_specs=pl.BlockSpec((r, 1), lambda i: (i, 0)),
        out_shape=jax.ShapeDtypeStruct((n_acc, 1), jnp.float32),
    )(q, dinv, b2p, W3p, b3r, W4, b4r)

    return out[:n]
